# baseline (device time: 640696 ns/iter reference)
import jax
import jax.numpy as jnp
import numpy as np
from jax import lax
from jax.experimental import pallas as pl
from jax.experimental.pallas import tpu as pltpu

N_DEV = 32
M = 2048
N = 1024
N_TREE = 2
NCHUNK = 16
CHUNK = M // N_TREE // NCHUNK
MAX_CH = 4
SEND_RING = 2
LAG_BASE = 4
LAG_PER_DEPTH = 2
MAX_DEPTH = 6
MAX_LAG = LAG_BASE + LAG_PER_DEPTH * MAX_DEPTH

_PLANE = [(0, 0), (1, 0), (1, 1), (0, 1), (0, 2), (1, 2), (1, 3), (0, 3)]
_POS2COORD = {}
_COORD2POS = {}
for _z in range(4):
    for _i, (_x, _y) in enumerate(_PLANE):
        _p = 8 * _z + _i
        _POS2COORD[_p] = (_x, _y, _z)
        _COORD2POS[(_x, _y, _z)] = _p

_S0 = {
    (0, 1): (1, 1), (1, 0): (1, 1), (1, 2): (1, 1),
    (0, 0): (1, 0), (2, 0): (1, 0),
    (3, 0): (2, 0),
    (0, 2): (1, 2), (2, 2): (1, 2), (1, 3): (1, 2),
    (0, 3): (0, 2),
    (3, 2): (2, 2), (2, 1): (2, 2),
    (3, 1): (2, 1),
    (3, 3): (3, 2),
    (2, 3): (1, 3),
}
_G1 = {
    (0, 3): (0, 2), (0, 1): (0, 2),
    (1, 2): (2, 2), (2, 3): (2, 2),
    (3, 2): (3, 3),
    (2, 1): (2, 0), (3, 0): (2, 0),
    (1, 0): (0, 0),
}


def _t1_parent(v):
    x, y, z = v
    if x == 0:
        if (y, z) == (1, 1):
            return None
        py, pz = _S0[(y, z)]
        return (0, py, pz)
    if (y + z) % 2 == 0:
        return (0, y, z)
    py, pz = _G1[(y, z)]
    return (1, py, pz)


def _tau(v):
    return (1 - v[0], v[1], 3 - v[2])


def _t2_parent(v):
    p = _t1_parent(_tau(v))
    return None if p is None else _tau(p)


def _tree_meta(parent_fn):
    par = np.full((N_DEV,), -1, np.int32)
    chl = np.full((N_DEV, MAX_CH), -1, np.int32)
    ncs = np.zeros((N_DEV,), np.int32)
    slot = np.zeros((N_DEV,), np.int32)
    for p in range(N_DEV):
        pc = parent_fn(_POS2COORD[p])
        if pc is None:
            continue
        pp = _COORD2POS[pc]
        par[p] = pp
        slot[p] = ncs[pp]
        chl[pp, ncs[pp]] = p
        ncs[pp] += 1
    assert int(ncs.max()) <= MAX_CH
    depth = np.zeros((N_DEV,), np.int32)
    for p in range(N_DEV):
        d, q = 0, p
        while par[q] >= 0:
            q = par[q]
            d += 1
        depth[p] = d
    assert int(depth.max()) <= MAX_DEPTH
    lag = (LAG_BASE + LAG_PER_DEPTH * depth).astype(np.int32)
    return np.concatenate(
        [par[:, None], ncs[:, None], slot[:, None], lag[:, None], chl], axis=1
    ).astype(np.int32)


_META = np.concatenate([_tree_meta(_t1_parent), _tree_meta(_t2_parent)], axis=1)
_TW = 4 + MAX_CH


def _f(s):
    return s * s * (jnp.tanh(s) + jnp.maximum(s, 0.0))


def kernel(t):
    my_pos = lax.axis_index("i")
    meta = jnp.asarray(_META)[my_pos]

    def body(meta_ref, x_ref, out_ref, up_buf,
             up_recv_sems, down_recv_sems, up_send_sems, down_send_sems):

        def ctx(tr):
            o = tr * _TW
            return dict(
                tr=tr,
                parent=meta_ref[o + 0],
                nc=meta_ref[o + 1],
                my_slot=meta_ref[o + 2],
                my_lag=meta_ref[o + 3],
                child=lambda j: meta_ref[o + 4 + j],
            )

        def rows(tr, c):
            return pl.ds(tr * (M // N_TREE) + c * CHUNK, CHUNK)

        def up_send_desc(cx, c):
            return pltpu.make_async_remote_copy(
                src_ref=out_ref.at[rows(cx["tr"], c), :],
                dst_ref=up_buf.at[cx["tr"], c, cx["my_slot"]],
                send_sem=up_send_sems.at[cx["tr"], c % SEND_RING],
                recv_sem=up_recv_sems.at[cx["tr"], c, cx["my_slot"]],
                device_id=(cx["parent"],),
                device_id_type=pl.DeviceIdType.MESH,
            )

        def up_recv_desc(cx, c, j):
            return pltpu.make_async_remote_copy(
                src_ref=up_buf.at[cx["tr"], c, j],
                dst_ref=up_buf.at[cx["tr"], c, j],
                send_sem=up_send_sems.at[cx["tr"], 0],
                recv_sem=up_recv_sems.at[cx["tr"], c, j],
                device_id=(cx["parent"],),
                device_id_type=pl.DeviceIdType.MESH,
            )

        def down_send_desc(cx, c, j, ring):
            return pltpu.make_async_remote_copy(
                src_ref=out_ref.at[rows(cx["tr"], c), :],
                dst_ref=out_ref.at[rows(cx["tr"], c), :],
                send_sem=down_send_sems.at[cx["tr"], ring, j],
                recv_sem=down_recv_sems.at[cx["tr"], c],
                device_id=(cx["child"](j),),
                device_id_type=pl.DeviceIdType.MESH,
            )

        def down_recv_desc(cx, c):
            return pltpu.make_async_remote_copy(
                src_ref=out_ref.at[rows(cx["tr"], c), :],
                dst_ref=out_ref.at[rows(cx["tr"], c), :],
                send_sem=up_send_sems.at[cx["tr"], 0],
                recv_sem=down_recv_sems.at[cx["tr"], c],
                device_id=(cx["parent"],),
                device_id_type=pl.DeviceIdType.MESH,
            )

        cxs = [ctx(0), ctx(1)]

        barrier_sem = pltpu.get_barrier_semaphore()
        n_expect = jnp.int32(0)
        for cx in cxs:
            has_parent = cx["parent"] >= 0
            n_expect = n_expect + cx["nc"] + jnp.where(has_parent, 1, 0)

            @pl.when(has_parent)
            def _():
                pl.semaphore_signal(
                    barrier_sem, inc=1,
                    device_id=(cx["parent"],),
                    device_id_type=pl.DeviceIdType.MESH,
                )

            for j in range(MAX_CH):
                @pl.when(j < cx["nc"])
                def _():
                    pl.semaphore_signal(
                        barrier_sem, inc=1,
                        device_id=(cx["child"](j),),
                        device_id_type=pl.DeviceIdType.MESH,
                    )
        pl.semaphore_wait(barrier_sem, n_expect)

        for i in range(NCHUNK + MAX_LAG):
            for cx in cxs:
                if i >= NCHUNK:
                    continue
                c = i
                tr = cx["tr"]
                nc = cx["nc"]
                has_parent = cx["parent"] >= 0
                for j in range(MAX_CH):
                    @pl.when(j < nc)
                    def _():
                        up_recv_desc(cx, c, j).wait_recv()

                for k in range(MAX_CH + 1):
                    @pl.when(nc == k)
                    def _():
                        acc = x_ref[rows(tr, c), :]
                        for j in range(k):
                            acc = acc + up_buf[tr, c, j]
                        out_ref[rows(tr, c), :] = acc

                @pl.when(has_parent)
                def _():
                    if c >= SEND_RING:
                        up_send_desc(cx, c - SEND_RING).wait_send()
                    up_send_desc(cx, c).start()

                @pl.when(jnp.logical_not(has_parent))
                def _():
                    out_ref[rows(tr, c), :] = _f(out_ref[rows(tr, c), :])
                    for j in range(MAX_CH):
                        @pl.when(j < nc)
                        def _():
                            if c >= SEND_RING:
                                down_send_desc(
                                    cx, c - SEND_RING, j,
                                    (c - SEND_RING) % SEND_RING,
                                ).wait_send()
                            down_send_desc(cx, c, j, c % SEND_RING).start()

            for cx in cxs:
                nc = cx["nc"]
                has_parent = cx["parent"] >= 0
                cd = i - cx["my_lag"]

                @pl.when(has_parent & (cd >= 0) & (cd < NCHUNK))
                def _():
                    down_recv_desc(cx, cd).wait_recv()
                    for j in range(MAX_CH):
                        @pl.when(j < nc)
                        def _():
                            @pl.when(cd >= SEND_RING)
                            def _():
                                down_send_desc(
                                    cx, cd - SEND_RING, j,
                                    lax.rem(cd - SEND_RING, SEND_RING),
                                ).wait_send()
                            down_send_desc(
                                cx, cd, j, lax.rem(cd, SEND_RING)
                            ).start()

        for cx in cxs:
            for c in range(NCHUNK - SEND_RING, NCHUNK):
                @pl.when(cx["parent"] >= 0)
                def _():
                    up_send_desc(cx, c).wait_send()
                for j in range(MAX_CH):
                    @pl.when(j < cx["nc"])
                    def _():
                        down_send_desc(cx, c, j, c % SEND_RING).wait_send()

    return pl.pallas_call(
        body,
        out_shape=jax.ShapeDtypeStruct((M, N), jnp.float32),
        in_specs=[
            pl.BlockSpec(memory_space=pltpu.SMEM),
            pl.BlockSpec(memory_space=pltpu.VMEM),
        ],
        out_specs=pl.BlockSpec(memory_space=pltpu.VMEM),
        scratch_shapes=[
            pltpu.VMEM((N_TREE, NCHUNK, MAX_CH, CHUNK, N), jnp.float32),
            pltpu.SemaphoreType.DMA((N_TREE, NCHUNK, MAX_CH)),
            pltpu.SemaphoreType.DMA((N_TREE, NCHUNK)),
            pltpu.SemaphoreType.DMA((N_TREE, SEND_RING)),
            pltpu.SemaphoreType.DMA((N_TREE, SEND_RING, MAX_CH)),
        ],
        compiler_params=pltpu.CompilerParams(
            collective_id=0,
            vmem_limit_bytes=100 * 1024 * 1024,
        ),
    )(meta, t)
